# Initial kernel scaffold; baseline (speedup 1.0000x reference)
import jax
import jax.numpy as jnp
from jax.experimental import pallas as pl


def kernel(features_indices, emb, W1, b1, W2, b2, W3, b3):
    raise NotImplementedError("write your pallas kernel here")


# speedup vs baseline: 10.4831x; 10.4831x over previous
import jax
import jax.numpy as jnp
from jax.experimental import pallas as pl

_F = 768
_E = 256
_H = 512
_BAG = 32
_BLK = 1024


def _nnue_block(idx_ref, emb_ref, w1_ref, b1_ref, w2_ref, b2_ref, w3_ref,
                b3_ref, out_ref):
    idx = idx_ref[...]
    iota = jax.lax.broadcasted_iota(jnp.int32, (_BLK, _F), 1)
    counts = jnp.zeros((_BLK, _F), jnp.int32)
    for j in range(_BAG):
        counts = counts + (idx[:, j:j + 1] == iota).astype(jnp.int32)
    x = jnp.dot(counts.astype(jnp.bfloat16), emb_ref[...],
                preferred_element_type=jnp.float32)
    h1 = jnp.dot(x.astype(jnp.bfloat16), w1_ref[...],
                 preferred_element_type=jnp.float32) + b1_ref[...]
    h1 = jnp.maximum(h1, 0.0).astype(jnp.bfloat16)
    h2 = jnp.dot(h1, w2_ref[...], preferred_element_type=jnp.float32) \
        + b2_ref[...]
    h2 = jnp.maximum(h2, 0.0)
    out_ref[...] = jnp.sum(h2 * w3_ref[...], axis=1, keepdims=True) \
        + b3_ref[...]


@jax.jit
def kernel(features_indices, emb, W1, b1, W2, b2, W3, b3):
    n = features_indices.shape[0]
    idx = features_indices.astype(jnp.int32)
    return pl.pallas_call(
        _nnue_block,
        grid=(n // _BLK,),
        in_specs=[
            pl.BlockSpec((_BLK, _BAG), lambda i: (i, 0)),
            pl.BlockSpec((_F, _E), lambda i: (0, 0)),
            pl.BlockSpec((_E, _H), lambda i: (0, 0)),
            pl.BlockSpec((1, _H), lambda i: (0, 0)),
            pl.BlockSpec((_H, _E), lambda i: (0, 0)),
            pl.BlockSpec((1, _E), lambda i: (0, 0)),
            pl.BlockSpec((1, _E), lambda i: (0, 0)),
            pl.BlockSpec((1, 1), lambda i: (0, 0)),
        ],
        out_specs=pl.BlockSpec((_BLK, 1), lambda i: (i, 0)),
        out_shape=jax.ShapeDtypeStruct((n, 1), jnp.float32),
    )(idx, emb.astype(jnp.bfloat16), W1.astype(jnp.bfloat16),
      b1.reshape(1, _H), W2.astype(jnp.bfloat16), b2.reshape(1, _E),
      W3.reshape(1, _E), b3.reshape(1, 1))


# speedup vs baseline: 11.1363x; 1.0623x over previous
import jax
import jax.numpy as jnp
from jax import lax
from jax.experimental import pallas as pl
from jax.experimental.pallas import tpu as pltpu
from jax.experimental.pallas import tpu_sc as plsc

_F = 768
_E = 256
_H = 512
_BAG = 32
_CHUNK = 4096
_NW = 32
_SPW = _CHUNK // _NW
_IDX_W = _SPW * _BAG
_CNT_W = _SPW * _F
_BLK = 1024


def _sc_hist_body(idx_hbm, cnt_hbm, idx_v, cnt_v):
    wid = lax.axis_index("s") * 2 + lax.axis_index("c")
    pltpu.sync_copy(idx_hbm.at[pl.ds(wid * _IDX_W, _IDX_W)], idx_v)

    @pl.loop(0, _CNT_W, step=16)
    def _zero(i):
        cnt_v[pl.ds(i, 16)] = jnp.zeros((16,), jnp.float32)

    ones = jnp.full((16,), 1.0, jnp.float32)

    @pl.loop(0, _SPW)
    def _hist(s):
        base = s * _BAG
        row = s * _F
        a = idx_v[pl.ds(base, 16)] + row
        b = idx_v[pl.ds(base + 16, 16)] + row
        plsc.addupdate_scatter(cnt_v, [a], ones)
        plsc.addupdate_scatter(cnt_v, [b], ones)

    pltpu.sync_copy(cnt_v, cnt_hbm.at[pl.ds(wid * _CNT_W, _CNT_W)])


import dataclasses
import functools


@functools.lru_cache(maxsize=1)
def _sc_hist():
    cp = pltpu.CompilerParams()
    if "needs_layout_passes" in pltpu.CompilerParams.__dataclass_fields__:
        cp = dataclasses.replace(cp, needs_layout_passes=False)
    return pl.kernel(
        _sc_hist_body,
        compiler_params=cp,
        out_type=jax.ShapeDtypeStruct((_CHUNK * _F,), jnp.float32),
        mesh=plsc.VectorSubcoreMesh(core_axis_name="c", subcore_axis_name="s"),
        scratch_types=[
            pltpu.VMEM((_IDX_W,), jnp.int32),
            pltpu.VMEM((_CNT_W,), jnp.float32),
        ],
    )


def _mlp_body(cnt_ref, emb_ref, w1_ref, b1_ref, w2_ref, b2_ref, w3_ref,
              b3_ref, out_ref):
    c = cnt_ref[...].astype(jnp.bfloat16)
    x = jnp.dot(c, emb_ref[...], preferred_element_type=jnp.float32)
    h1 = jnp.dot(x.astype(jnp.bfloat16), w1_ref[...],
                 preferred_element_type=jnp.float32) + b1_ref[...]
    h1 = jnp.maximum(h1, 0.0).astype(jnp.bfloat16)
    h2 = jnp.dot(h1, w2_ref[...], preferred_element_type=jnp.float32) \
        + b2_ref[...]
    h2 = jnp.maximum(h2, 0.0)
    out_ref[...] = jnp.sum(h2 * w3_ref[...], axis=1, keepdims=True) \
        + b3_ref[...]


def _mlp_chunk(counts, embb, w1b, b1r, w2b, b2r, w3r, b3r):
    return pl.pallas_call(
        _mlp_body,
        grid=(_CHUNK // _BLK,),
        in_specs=[
            pl.BlockSpec((_BLK, _F), lambda i: (i, 0)),
            pl.BlockSpec((_F, _E), lambda i: (0, 0)),
            pl.BlockSpec((_E, _H), lambda i: (0, 0)),
            pl.BlockSpec((1, _H), lambda i: (0, 0)),
            pl.BlockSpec((_H, _E), lambda i: (0, 0)),
            pl.BlockSpec((1, _E), lambda i: (0, 0)),
            pl.BlockSpec((1, _E), lambda i: (0, 0)),
            pl.BlockSpec((1, 1), lambda i: (0, 0)),
        ],
        out_specs=pl.BlockSpec((_BLK, 1), lambda i: (i, 0)),
        out_shape=jax.ShapeDtypeStruct((_CHUNK, 1), jnp.float32),
    )(counts, embb, w1b, b1r, w2b, b2r, w3r, b3r)


@jax.jit
def kernel(features_indices, emb, W1, b1, W2, b2, W3, b3):
    n = features_indices.shape[0]
    idx = features_indices.astype(jnp.int32).reshape(-1)
    embb = emb.astype(jnp.bfloat16)
    w1b = W1.astype(jnp.bfloat16)
    w2b = W2.astype(jnp.bfloat16)
    b1r = b1.reshape(1, _H)
    b2r = b2.reshape(1, _E)
    w3r = W3.reshape(1, _E)
    b3r = b3.reshape(1, 1)
    outs = []
    for c in range(n // _CHUNK):
        idx_c = lax.slice(idx, (c * _CHUNK * _BAG,), ((c + 1) * _CHUNK * _BAG,))
        counts = _sc_hist()(idx_c).reshape(_CHUNK, _F)
        outs.append(_mlp_chunk(counts, embb, w1b, b1r, w2b, b2r, w3r, b3r))
    return jnp.concatenate(outs, axis=0)


# speedup vs baseline: 13.4152x; 1.2046x over previous
import jax
import jax.numpy as jnp
from jax import lax
from jax.experimental import pallas as pl
from jax.experimental.pallas import tpu as pltpu
from jax.experimental.pallas import tpu_sc as plsc

_F = 768
_E = 256
_H = 512
_BAG = 32
_CHUNK = 4096
_NW = 32
_SPW = _CHUNK // _NW
_IDX_W = _SPW * _BAG
_CNT_W = _SPW * _F
_BLK = 1024


_GRP = 32
_NGRP = _SPW // _GRP
_GRP_W = _GRP * _F


def _sc_hist_body(idx_hbm, cnt_hbm, idx_v, cnt_a, cnt_b, sem_a, sem_b):
    wid = lax.axis_index("s") * 2 + lax.axis_index("c")
    pltpu.sync_copy(idx_hbm.at[pl.ds(wid * _IDX_W, _IDX_W)], idx_v)

    @pl.loop(0, _GRP_W, step=16)
    def _zero(i):
        z = jnp.zeros((16,), jnp.float32)
        cnt_a[pl.ds(i, 16)] = z
        cnt_b[pl.ds(i, 16)] = z

    ones = jnp.full((16,), 1.0, jnp.float32)
    zeros = jnp.zeros((16,), jnp.float32)
    bufs = [(cnt_a, sem_a), (cnt_b, sem_b)]
    copies = [None, None]

    for g in range(_NGRP):
        cnt, sem = bufs[g % 2]
        if copies[g % 2] is not None:
            copies[g % 2].wait()
            pg = g - 2

            def _rezero(t, pg=pg, cnt=cnt):
                base = (pg * _GRP + t) * _BAG
                row = t * _F
                a = idx_v[pl.ds(base, 16)] + row
                b = idx_v[pl.ds(base + 16, 16)] + row
                plsc.store_scatter(cnt, [a], zeros)
                plsc.store_scatter(cnt, [b], zeros)

            pl.loop(0, _GRP)(_rezero)

        def _hist(t, g=g, cnt=cnt):
            base = (g * _GRP + t) * _BAG
            row = t * _F
            a = idx_v[pl.ds(base, 16)] + row
            b = idx_v[pl.ds(base + 16, 16)] + row
            plsc.addupdate_scatter(cnt, [a], ones)
            plsc.addupdate_scatter(cnt, [b], ones)

        pl.loop(0, _GRP)(_hist)
        copies[g % 2] = pltpu.async_copy(
            cnt, cnt_hbm.at[pl.ds(wid * _CNT_W + g * _GRP_W, _GRP_W)], sem)

    copies[(_NGRP - 2) % 2].wait()
    copies[(_NGRP - 1) % 2].wait()


import dataclasses
import functools


@functools.lru_cache(maxsize=1)
def _sc_hist():
    cp = pltpu.CompilerParams()
    if "needs_layout_passes" in pltpu.CompilerParams.__dataclass_fields__:
        cp = dataclasses.replace(cp, needs_layout_passes=False)
    return pl.kernel(
        _sc_hist_body,
        compiler_params=cp,
        out_type=jax.ShapeDtypeStruct((_CHUNK * _F,), jnp.float32),
        mesh=plsc.VectorSubcoreMesh(core_axis_name="c", subcore_axis_name="s"),
        scratch_types=[
            pltpu.VMEM((_IDX_W,), jnp.int32),
            pltpu.VMEM((_GRP_W,), jnp.float32),
            pltpu.VMEM((_GRP_W,), jnp.float32),
            pltpu.SemaphoreType.DMA,
            pltpu.SemaphoreType.DMA,
        ],
    )


def _mlp_body(cnt_ref, emb_ref, w1_ref, b1_ref, w2_ref, b2_ref, w3_ref,
              b3_ref, out_ref):
    c = cnt_ref[...].astype(jnp.bfloat16)
    x = jnp.dot(c, emb_ref[...], preferred_element_type=jnp.float32)
    h1 = jnp.dot(x.astype(jnp.bfloat16), w1_ref[...],
                 preferred_element_type=jnp.float32) + b1_ref[...]
    h1 = jnp.maximum(h1, 0.0).astype(jnp.bfloat16)
    h2 = jnp.dot(h1, w2_ref[...], preferred_element_type=jnp.float32) \
        + b2_ref[...]
    h2 = jnp.maximum(h2, 0.0)
    out_ref[...] = jnp.sum(h2 * w3_ref[...], axis=1, keepdims=True) \
        + b3_ref[...]


def _mlp_chunk(counts, embb, w1b, b1r, w2b, b2r, w3r, b3r):
    return pl.pallas_call(
        _mlp_body,
        grid=(_CHUNK // _BLK,),
        in_specs=[
            pl.BlockSpec((_BLK, _F), lambda i: (i, 0)),
            pl.BlockSpec((_F, _E), lambda i: (0, 0)),
            pl.BlockSpec((_E, _H), lambda i: (0, 0)),
            pl.BlockSpec((1, _H), lambda i: (0, 0)),
            pl.BlockSpec((_H, _E), lambda i: (0, 0)),
            pl.BlockSpec((1, _E), lambda i: (0, 0)),
            pl.BlockSpec((1, _E), lambda i: (0, 0)),
            pl.BlockSpec((1, 1), lambda i: (0, 0)),
        ],
        out_specs=pl.BlockSpec((_BLK, 1), lambda i: (i, 0)),
        out_shape=jax.ShapeDtypeStruct((_CHUNK, 1), jnp.float32),
    )(counts, embb, w1b, b1r, w2b, b2r, w3r, b3r)


@jax.jit
def kernel(features_indices, emb, W1, b1, W2, b2, W3, b3):
    n = features_indices.shape[0]
    idx = features_indices.astype(jnp.int32).reshape(-1)
    embb = emb.astype(jnp.bfloat16)
    w1b = W1.astype(jnp.bfloat16)
    w2b = W2.astype(jnp.bfloat16)
    b1r = b1.reshape(1, _H)
    b2r = b2.reshape(1, _E)
    w3r = W3.reshape(1, _E)
    b3r = b3.reshape(1, 1)
    outs = []
    for c in range(n // _CHUNK):
        idx_c = lax.slice(idx, (c * _CHUNK * _BAG,), ((c + 1) * _CHUNK * _BAG,))
        counts = _sc_hist()(idx_c).reshape(_CHUNK, _F)
        outs.append(_mlp_chunk(counts, embb, w1b, b1r, w2b, b2r, w3r, b3r))
    return jnp.concatenate(outs, axis=0)


# speedup vs baseline: 21.0647x; 1.5702x over previous
import dataclasses
import functools

import jax
import jax.numpy as jnp
from jax import lax
from jax.experimental import pallas as pl
from jax.experimental.pallas import tpu as pltpu
from jax.experimental.pallas import tpu_sc as plsc

_F = 768
_E = 256
_H = 512
_BAG = 32
_CHUNK = 4096
_NW = 32
_SPW = _CHUNK // _NW
_GRP = 32
_NGRP = _SPW // _GRP
_BLK = 1024


def _sc_hist_body(chunk_row0, idx_hbm, cnt_hbm, idx_v, cnt_a, cnt_b,
                  sem_a, sem_b):
    wid = lax.axis_index("s") * 2 + lax.axis_index("c")
    row0 = wid * _SPW
    pltpu.sync_copy(idx_hbm.at[pl.ds(chunk_row0 + row0, _SPW)], idx_v)

    @pl.loop(0, _GRP)
    def _zero_r(r):
        @pl.loop(0, _F, step=16)
        def _zero_c(i):
            z = jnp.zeros((16,), jnp.float32)
            cnt_a[r, pl.ds(i, 16)] = z
            cnt_b[r, pl.ds(i, 16)] = z

    ones = jnp.full((16,), 1.0, jnp.float32)
    zeros = jnp.zeros((16,), jnp.float32)
    bufs = [(cnt_a, sem_a), (cnt_b, sem_b)]
    copies = [None, None]

    for g in range(_NGRP):
        cnt, sem = bufs[g % 2]
        if copies[g % 2] is not None:
            copies[g % 2].wait()
            pg = g - 2

            def _rezero(t, pg=pg, cnt=cnt):
                rows = jnp.full((16,), t, jnp.int32)
                a = idx_v[pg * _GRP + t, pl.ds(0, 16)]
                b = idx_v[pg * _GRP + t, pl.ds(16, 16)]
                plsc.store_scatter(cnt, [rows, a], zeros)
                plsc.store_scatter(cnt, [rows, b], zeros)

            pl.loop(0, _GRP)(_rezero)

        def _hist(t, g=g, cnt=cnt):
            rows = jnp.full((16,), t, jnp.int32)
            a = idx_v[g * _GRP + t, pl.ds(0, 16)]
            b = idx_v[g * _GRP + t, pl.ds(16, 16)]
            plsc.addupdate_scatter(cnt, [rows, a], ones)
            plsc.addupdate_scatter(cnt, [rows, b], ones)

        pl.loop(0, _GRP)(_hist)
        copies[g % 2] = pltpu.async_copy(
            cnt, cnt_hbm.at[pl.ds(row0 + g * _GRP, _GRP)], sem)

    copies[(_NGRP - 2) % 2].wait()
    copies[(_NGRP - 1) % 2].wait()


@functools.lru_cache(maxsize=None)
def _sc_hist(chunk_row0):
    cp = pltpu.CompilerParams()
    if "needs_layout_passes" in pltpu.CompilerParams.__dataclass_fields__:
        cp = dataclasses.replace(cp, needs_layout_passes=False)
    return pl.kernel(
        functools.partial(_sc_hist_body, chunk_row0),
        compiler_params=cp,
        out_type=jax.ShapeDtypeStruct((_CHUNK, _F), jnp.float32),
        mesh=plsc.VectorSubcoreMesh(core_axis_name="c", subcore_axis_name="s"),
        scratch_types=[
            pltpu.VMEM((_SPW, _BAG), jnp.int32),
            pltpu.VMEM((_GRP, _F), jnp.float32),
            pltpu.VMEM((_GRP, _F), jnp.float32),
            pltpu.SemaphoreType.DMA,
            pltpu.SemaphoreType.DMA,
        ],
    )


def _mlp_body(cnt_ref, emb_ref, w1_ref, b1_ref, w2_ref, b2_ref, w3_ref,
              b3_ref, out_ref):
    c = cnt_ref[...].astype(jnp.bfloat16)
    x = jnp.dot(c, emb_ref[...], preferred_element_type=jnp.float32)
    h1 = jnp.dot(x.astype(jnp.bfloat16), w1_ref[...],
                 preferred_element_type=jnp.float32) + b1_ref[...]
    h1 = jnp.maximum(h1, 0.0).astype(jnp.bfloat16)
    h2 = jnp.dot(h1, w2_ref[...], preferred_element_type=jnp.float32) \
        + b2_ref[...]
    h2 = jnp.maximum(h2, 0.0)
    out_ref[...] = jnp.sum(h2 * w3_ref[...], axis=1, keepdims=True) \
        + b3_ref[...]


def _mlp_chunk(counts, embb, w1b, b1r, w2b, b2r, w3r, b3r):
    return pl.pallas_call(
        _mlp_body,
        grid=(_CHUNK // _BLK,),
        in_specs=[
            pl.BlockSpec((_BLK, _F), lambda i: (i, 0)),
            pl.BlockSpec((_F, _E), lambda i: (0, 0)),
            pl.BlockSpec((_E, _H), lambda i: (0, 0)),
            pl.BlockSpec((1, _H), lambda i: (0, 0)),
            pl.BlockSpec((_H, _E), lambda i: (0, 0)),
            pl.BlockSpec((1, _E), lambda i: (0, 0)),
            pl.BlockSpec((1, _E), lambda i: (0, 0)),
            pl.BlockSpec((1, 1), lambda i: (0, 0)),
        ],
        out_specs=pl.BlockSpec((_BLK, 1), lambda i: (i, 0)),
        out_shape=jax.ShapeDtypeStruct((_CHUNK, 1), jnp.float32),
    )(counts, embb, w1b, b1r, w2b, b2r, w3r, b3r)


@jax.jit
def kernel(features_indices, emb, W1, b1, W2, b2, W3, b3):
    n = features_indices.shape[0]
    idx = features_indices.astype(jnp.int32)
    embb = emb.astype(jnp.bfloat16)
    w1b = W1.astype(jnp.bfloat16)
    w2b = W2.astype(jnp.bfloat16)
    b1r = b1.reshape(1, _H)
    b2r = b2.reshape(1, _E)
    w3r = W3.reshape(1, _E)
    b3r = b3.reshape(1, 1)
    outs = []
    for c in range(n // _CHUNK):
        counts = _sc_hist(c * _CHUNK)(idx)
        outs.append(_mlp_chunk(counts, embb, w1b, b1r, w2b, b2r, w3r, b3r))
    return jnp.concatenate(outs, axis=0)
